# baseline (speedup 1.0000x reference)
import functools

import jax
import jax.numpy as jnp
from jax import lax
from jax.experimental import pallas as pl
from jax.experimental.pallas import tpu as pltpu
from jax.experimental.pallas import tpu_sc as plsc

_N = 10000
_D = 128
_OUT = 26
_L = 4
_E = 320000

_NP = 10240
_RPT = _NP // 16
_EP = 327680
_EPT = _EP // 32
_C = 128
_NCHUNK = _EPT // _C
_NB = 2

_mesh = plsc.VectorSubcoreMesh(core_axis_name="c", subcore_axis_name="s")


def _fill(ref, rows, cols, val):
    v = jnp.full((16,), val, jnp.float32)

    def row(i, carry):
        for j in range(cols // 16):
            ref[i, pl.ds(j * 16, 16)] = v
        return carry

    lax.fori_loop(0, rows, row, 0)


def _make_agg():
    out_type = [jax.ShapeDtypeStruct((2 * _NP, _D), jnp.float32)]
    scratch = [
        pltpu.VMEM((_NB, _C), jnp.int32),
        pltpu.VMEM((_NCHUNK, _C), jnp.int32),
        pltpu.VMEM((_NB, _C, _D), jnp.float32),
        pltpu.VMEM_SHARED((_NP, _D), jnp.float32),
    ] + [pltpu.SemaphoreType.DMA] * (3 * _NB)

    @functools.partial(pl.kernel, out_type=out_type, mesh=_mesh,
                       scratch_types=scratch,
                       compiler_params=pltpu.CompilerParams(
                           needs_layout_passes=False))
    def agg(h_hbm, src_hbm, dst_hbm, out_h, src_c, dst_l, rows, acc_h,
            *sems):
        gsems = sems[:_NB]
        isems = sems[_NB:2 * _NB]
        ssems = sems[2 * _NB:]
        cid = lax.axis_index("c")
        sid = lax.axis_index("s")
        wid = sid * 2 + cid

        c0 = pl.multiple_of(wid * _NCHUNK, _NCHUNK)
        pltpu.sync_copy(dst_hbm.at[pl.ds(c0, _NCHUNK)], dst_l)

        _fill(rows.at[0], _C, _D, 0.0)
        r0 = pl.multiple_of(sid * _RPT, _RPT)
        for k in range(_RPT // _C):
            pltpu.sync_copy(rows.at[0], acc_h.at[pl.ds(r0 + k * _C, _C)])

        for b in range(_NB):
            pltpu.sync_copy(src_hbm.at[c0 + b], src_c.at[b])
        pltpu.async_copy(h_hbm.at[src_c.at[0]], rows.at[0], gsems[0])
        plsc.subcore_barrier()

        def visit(j, b, fire_idx, fire_gather, wait_sc, wait_idx):
            b2 = b ^ 1
            pltpu.make_async_copy(h_hbm.at[src_c.at[b]], rows.at[b],
                                  gsems[b]).wait()
            if fire_idx:
                pltpu.async_copy(src_hbm.at[c0 + j + 2], src_c.at[b],
                                 isems[b])
            pltpu.async_copy(rows.at[b], acc_h.at[dst_l.at[j]], ssems[b],
                             add=True)
            if wait_sc:
                pltpu.make_async_copy(rows.at[b2],
                                      acc_h.at[dst_l.at[j - 1]],
                                      ssems[b2]).wait()
            if fire_gather:
                if wait_idx:
                    pltpu.make_async_copy(src_hbm.at[c0 + j + 1],
                                          src_c.at[b2], isems[b2]).wait()
                pltpu.async_copy(h_hbm.at[src_c.at[b2]], rows.at[b2],
                                 gsems[b2])

        visit(0, 0, True, True, False, False)

        def pair(g, carry):
            visit(1 + g * 2, 1, True, True, True, True)
            visit(2 + g * 2, 0, True, True, True, True)
            return carry

        lax.fori_loop(0, (_NCHUNK - 4) // 2, pair, 0)
        visit(_NCHUNK - 3, (_NCHUNK - 3) % 2, True, True, True, True)
        visit(_NCHUNK - 2, (_NCHUNK - 2) % 2, False, True, True, True)
        visit(_NCHUNK - 1, (_NCHUNK - 1) % 2, False, False, True, False)
        pltpu.make_async_copy(rows.at[(_NCHUNK - 1) % 2],
                              acc_h.at[dst_l.at[_NCHUNK - 1]],
                              ssems[(_NCHUNK - 1) % 2]).wait()
        plsc.subcore_barrier()

        o0 = pl.multiple_of(cid * _NP + sid * _RPT, _RPT)
        pltpu.sync_copy(acc_h.at[pl.ds(r0, _RPT)], out_h.at[pl.ds(o0, _RPT)])

    return agg


def _make_deg():
    out_type = [jax.ShapeDtypeStruct((2 * _NP,), jnp.float32)]
    scratch = [
        pltpu.VMEM((_NCHUNK, _C), jnp.int32),
        pltpu.VMEM((_NP,), jnp.float32),
        pltpu.VMEM((16, _RPT), jnp.float32),
        pltpu.VMEM_SHARED((16, _NP), jnp.float32),
    ]

    @functools.partial(pl.kernel, out_type=out_type, mesh=_mesh,
                       scratch_types=scratch,
                       compiler_params=pltpu.CompilerParams(
                           needs_layout_passes=False))
    def deg(dst_hbm, out_d, dst_l, deg_l, stage, deg_sh):
        cid = lax.axis_index("c")
        sid = lax.axis_index("s")
        wid = sid * 2 + cid

        c0 = pl.multiple_of(wid * _NCHUNK, _NCHUNK)
        pltpu.sync_copy(dst_hbm.at[pl.ds(c0, _NCHUNK)], dst_l)
        z = jnp.zeros((16,), jnp.float32)

        def zrow(i, carry):
            deg_l[pl.ds(i * 16, 16)] = z
            return carry

        lax.fori_loop(0, _NP // 16, zrow, 0)
        ones = jnp.ones((16,), jnp.float32)

        def chunk(j, carry):
            for k in range(_C // 16):
                iv = dst_l[j, pl.ds(k * 16, 16)]
                plsc.addupdate_scatter(deg_l, [iv], ones)
            return carry

        lax.fori_loop(0, _NCHUNK, chunk, 0)
        pltpu.sync_copy(deg_l, deg_sh.at[sid])
        plsc.subcore_barrier()
        r0 = pl.multiple_of(sid * _RPT, _RPT)
        for t in range(16):
            pltpu.sync_copy(deg_sh.at[t, pl.ds(r0, _RPT)], stage.at[t])

        def srow(i, carry):
            s = jnp.zeros((16,), jnp.float32)
            for t in range(16):
                s = s + stage[t, pl.ds(i * 16, 16)]
            deg_l[pl.ds(i * 16, 16)] = s
            return carry

        lax.fori_loop(0, _RPT // 16, srow, 0)
        o0 = pl.multiple_of(cid * _NP + sid * _RPT, _RPT)
        pltpu.sync_copy(deg_l.at[pl.ds(0, _RPT)], out_d.at[pl.ds(o0, _RPT)])

    return deg


_agg = _make_agg()
_deg = _make_deg()

_BR = 640


def _mid_body(h_ref, p_ref, d_ref, ws_ref, wn_ref, b_ref, o_ref):
    deg = jnp.maximum(d_ref[0] + d_ref[1], 1.0)
    neigh = (p_ref[0] + p_ref[1]) * (1.0 / deg)[:, None]
    acc = jnp.dot(h_ref[...], ws_ref[...], preferred_element_type=jnp.float32)
    acc = acc + jnp.dot(neigh, wn_ref[...],
                        preferred_element_type=jnp.float32)
    o_ref[...] = jnp.maximum(acc + b_ref[...], 0.0)


def _last_body(h_ref, p_ref, d_ref, ws_ref, wn_ref, b_ref, wp_ref, bp_ref,
               o_ref):
    deg = jnp.maximum(d_ref[0] + d_ref[1], 1.0)
    neigh = (p_ref[0] + p_ref[1]) * (1.0 / deg)[:, None]
    acc = jnp.dot(h_ref[...], ws_ref[...], preferred_element_type=jnp.float32)
    acc = acc + jnp.dot(neigh, wn_ref[...],
                        preferred_element_type=jnp.float32)
    h4 = jnp.maximum(acc + b_ref[...], 0.0)
    o_ref[...] = jnp.dot(h4, wp_ref[...],
                         preferred_element_type=jnp.float32) + bp_ref[...]


_WSPEC = [
    pl.BlockSpec((_D, _D), lambda i: (0, 0)),
    pl.BlockSpec((_D, _D), lambda i: (0, 0)),
    pl.BlockSpec((1, _D), lambda i: (0, 0)),
]
_HSPEC = pl.BlockSpec((_BR, _D), lambda i: (i, 0))
_PSPEC = pl.BlockSpec((2, _BR, _D), lambda i: (0, i, 0))
_DSPEC = pl.BlockSpec((2, _BR), lambda i: (0, i))


def _mid_layer_tc(h, p, d, ws, wn, b):
    return pl.pallas_call(
        _mid_body,
        grid=(_NP // _BR,),
        in_specs=[_HSPEC, _PSPEC, _DSPEC] + _WSPEC,
        out_specs=_HSPEC,
        out_shape=jax.ShapeDtypeStruct((_NP, _D), jnp.float32),
    )(h, p, d, ws, wn, b)


def _last_layer_tc(h, p, d, ws, wn, b, wp, bp):
    return pl.pallas_call(
        _last_body,
        grid=(_NP // _BR,),
        in_specs=[_HSPEC, _PSPEC, _DSPEC] + _WSPEC + [
            pl.BlockSpec((_D, _D), lambda i: (0, 0)),
            pl.BlockSpec((1, _D), lambda i: (0, 0)),
        ],
        out_specs=_HSPEC,
        out_shape=jax.ShapeDtypeStruct((_NP, _D), jnp.float32),
    )(h, p, d, ws, wn, b, wp, bp)


def kernel(x, edge_index, Ws, Wn, bs, Wp, bp):
    src = edge_index[0]
    dst = edge_index[1]
    pad_e = _EP - _E
    pad_i = jnp.arange(pad_e, dtype=jnp.int32)
    src_p = jnp.concatenate([src, pad_i % _N]).reshape(_EP // _C, _C)
    dst_p = jnp.concatenate([dst, _N + pad_i % (_NP - _N)]).reshape(
        _EP // _C, _C)
    h = jnp.zeros((_NP, _D), jnp.float32).at[:_N].set(x)

    wp_pad = jnp.zeros((_D, _D), jnp.float32).at[:, :_OUT].set(Wp)
    bp_pad = jnp.zeros((1, _D), jnp.float32).at[0, :_OUT].set(bp)

    (pd_flat,) = _deg(dst_p)
    pd = pd_flat.reshape(2, _NP)
    out = None
    for i in range(_L):
        (ph,) = _agg(h, src_p, dst_p)
        p = ph.reshape(2, _NP, _D)
        ws = Ws[i]
        wn = Wn[i]
        b = bs[i].reshape(1, _D)
        if i < _L - 1:
            h = _mid_layer_tc(h, p, pd, ws, wn, b)
        else:
            out = _last_layer_tc(h, p, pd, ws, wn, b, wp_pad, bp_pad)
    return out[:_N, :_OUT]


# speedup vs baseline: 1.1746x; 1.1746x over previous
import functools

import jax
import jax.numpy as jnp
from jax import lax
from jax.experimental import pallas as pl
from jax.experimental.pallas import tpu as pltpu
from jax.experimental.pallas import tpu_sc as plsc

_N = 10000
_D = 128
_OUT = 26
_L = 4
_E = 320000

_NP = 10240
_RPT = _NP // 16
_EP = 327680
_EPT = _EP // 32
_C = 128
_NCHUNK = _EPT // _C
_NB = 2

_mesh = plsc.VectorSubcoreMesh(core_axis_name="c", subcore_axis_name="s")


def _fill(ref, rows, cols, val):
    v = jnp.full((16,), val, jnp.float32)

    def row(i, carry):
        for j in range(cols // 16):
            ref[i, pl.ds(j * 16, 16)] = v
        return carry

    lax.fori_loop(0, rows, row, 0)


def _make_agg():
    out_type = [jax.ShapeDtypeStruct((2 * _NP, _D), jnp.float32)]
    scratch = [
        pltpu.VMEM((_NB, _C), jnp.int32),
        pltpu.VMEM((_NCHUNK, _C), jnp.int32),
        pltpu.VMEM((_NB, _C, _D), jnp.float32),
        pltpu.VMEM_SHARED((_NP, _D), jnp.float32),
    ] + [pltpu.SemaphoreType.DMA] * (2 * _NB)

    @functools.partial(pl.kernel, out_type=out_type, mesh=_mesh,
                       scratch_types=scratch,
                       compiler_params=pltpu.CompilerParams(
                           needs_layout_passes=False))
    def agg(h_hbm, src_hbm, dst_hbm, out_h, src_c, dst_l, rows, acc_h,
            *sems):
        gsems = sems[:_NB]
        isems = sems[_NB:2 * _NB]
        cid = lax.axis_index("c")
        sid = lax.axis_index("s")
        wid = sid * 2 + cid

        c0 = pl.multiple_of(wid * _NCHUNK, _NCHUNK)
        pltpu.sync_copy(dst_hbm.at[pl.ds(c0, _NCHUNK)], dst_l)

        _fill(rows.at[0], _C, _D, 0.0)
        r0 = pl.multiple_of(sid * _RPT, _RPT)
        for k in range(_RPT // _C):
            pltpu.sync_copy(rows.at[0], acc_h.at[pl.ds(r0 + k * _C, _C)])

        for b in range(2):
            pltpu.sync_copy(src_hbm.at[c0 + b], src_c.at[b])
            pltpu.async_copy(h_hbm.at[src_c.at[b]], rows.at[b], gsems[b])
        plsc.subcore_barrier()

        def do_chunk(j, b, fire_next):
            pltpu.make_async_copy(h_hbm.at[src_c.at[b]], rows.at[b],
                                  gsems[b]).wait()
            if fire_next:
                pltpu.async_copy(src_hbm.at[c0 + j + 2], src_c.at[b],
                                 isems[b])
            pltpu.sync_copy(rows.at[b], acc_h.at[dst_l.at[j]], add=True)
            if fire_next:
                pltpu.make_async_copy(src_hbm.at[c0 + j + 2], src_c.at[b],
                                      isems[b]).wait()
                pltpu.async_copy(h_hbm.at[src_c.at[b]], rows.at[b], gsems[b])

        def pair(g, carry):
            for b in range(2):
                do_chunk(g * 2 + b, b, True)
            return carry

        lax.fori_loop(0, _NCHUNK // 2 - 1, pair, 0)
        for b in range(2):
            do_chunk(_NCHUNK - 2 + b, b, False)
        plsc.subcore_barrier()

        o0 = pl.multiple_of(cid * _NP + sid * _RPT, _RPT)
        pltpu.sync_copy(acc_h.at[pl.ds(r0, _RPT)], out_h.at[pl.ds(o0, _RPT)])

    return agg


def _make_deg():
    out_type = [jax.ShapeDtypeStruct((2 * _NP,), jnp.float32)]
    scratch = [
        pltpu.VMEM((_NCHUNK, _C), jnp.int32),
        pltpu.VMEM((_NP,), jnp.float32),
        pltpu.VMEM((16, _RPT), jnp.float32),
        pltpu.VMEM_SHARED((16, _NP), jnp.float32),
    ]

    @functools.partial(pl.kernel, out_type=out_type, mesh=_mesh,
                       scratch_types=scratch,
                       compiler_params=pltpu.CompilerParams(
                           needs_layout_passes=False))
    def deg(dst_hbm, out_d, dst_l, deg_l, stage, deg_sh):
        cid = lax.axis_index("c")
        sid = lax.axis_index("s")
        wid = sid * 2 + cid

        c0 = pl.multiple_of(wid * _NCHUNK, _NCHUNK)
        pltpu.sync_copy(dst_hbm.at[pl.ds(c0, _NCHUNK)], dst_l)
        z = jnp.zeros((16,), jnp.float32)

        def zrow(i, carry):
            deg_l[pl.ds(i * 16, 16)] = z
            return carry

        lax.fori_loop(0, _NP // 16, zrow, 0)
        ones = jnp.ones((16,), jnp.float32)

        def chunk(j, carry):
            for k in range(_C // 16):
                iv = dst_l[j, pl.ds(k * 16, 16)]
                plsc.addupdate_scatter(deg_l, [iv], ones)
            return carry

        lax.fori_loop(0, _NCHUNK, chunk, 0)
        pltpu.sync_copy(deg_l, deg_sh.at[sid])
        plsc.subcore_barrier()
        r0 = pl.multiple_of(sid * _RPT, _RPT)
        for t in range(16):
            pltpu.sync_copy(deg_sh.at[t, pl.ds(r0, _RPT)], stage.at[t])

        def srow(i, carry):
            s = jnp.zeros((16,), jnp.float32)
            for t in range(16):
                s = s + stage[t, pl.ds(i * 16, 16)]
            deg_l[pl.ds(i * 16, 16)] = s
            return carry

        lax.fori_loop(0, _RPT // 16, srow, 0)
        o0 = pl.multiple_of(cid * _NP + sid * _RPT, _RPT)
        pltpu.sync_copy(deg_l.at[pl.ds(0, _RPT)], out_d.at[pl.ds(o0, _RPT)])

    return deg


_agg = _make_agg()
_deg = _make_deg()

_BR = 640


def _mid_body(h_ref, p_ref, d_ref, ws_ref, wn_ref, b_ref, o_ref):
    deg = jnp.maximum(d_ref[0] + d_ref[1], 1.0)
    neigh = (p_ref[0] + p_ref[1]) * (1.0 / deg)[:, None]
    acc = jnp.dot(h_ref[...], ws_ref[...], preferred_element_type=jnp.float32)
    acc = acc + jnp.dot(neigh, wn_ref[...],
                        preferred_element_type=jnp.float32)
    o_ref[...] = jnp.maximum(acc + b_ref[...], 0.0)


def _last_body(h_ref, p_ref, d_ref, ws_ref, wn_ref, b_ref, wp_ref, bp_ref,
               o_ref):
    deg = jnp.maximum(d_ref[0] + d_ref[1], 1.0)
    neigh = (p_ref[0] + p_ref[1]) * (1.0 / deg)[:, None]
    acc = jnp.dot(h_ref[...], ws_ref[...], preferred_element_type=jnp.float32)
    acc = acc + jnp.dot(neigh, wn_ref[...],
                        preferred_element_type=jnp.float32)
    h4 = jnp.maximum(acc + b_ref[...], 0.0)
    o_ref[...] = jnp.dot(h4, wp_ref[...],
                         preferred_element_type=jnp.float32) + bp_ref[...]


_WSPEC = [
    pl.BlockSpec((_D, _D), lambda i: (0, 0)),
    pl.BlockSpec((_D, _D), lambda i: (0, 0)),
    pl.BlockSpec((1, _D), lambda i: (0, 0)),
]
_HSPEC = pl.BlockSpec((_BR, _D), lambda i: (i, 0))
_PSPEC = pl.BlockSpec((2, _BR, _D), lambda i: (0, i, 0))
_DSPEC = pl.BlockSpec((2, _BR), lambda i: (0, i))


def _mid_layer_tc(h, p, d, ws, wn, b):
    return pl.pallas_call(
        _mid_body,
        grid=(_NP // _BR,),
        in_specs=[_HSPEC, _PSPEC, _DSPEC] + _WSPEC,
        out_specs=_HSPEC,
        out_shape=jax.ShapeDtypeStruct((_NP, _D), jnp.float32),
    )(h, p, d, ws, wn, b)


def _last_layer_tc(h, p, d, ws, wn, b, wp, bp):
    return pl.pallas_call(
        _last_body,
        grid=(_NP // _BR,),
        in_specs=[_HSPEC, _PSPEC, _DSPEC] + _WSPEC + [
            pl.BlockSpec((_D, _D), lambda i: (0, 0)),
            pl.BlockSpec((1, _D), lambda i: (0, 0)),
        ],
        out_specs=_HSPEC,
        out_shape=jax.ShapeDtypeStruct((_NP, _D), jnp.float32),
    )(h, p, d, ws, wn, b, wp, bp)


def kernel(x, edge_index, Ws, Wn, bs, Wp, bp):
    src = edge_index[0]
    dst = edge_index[1]
    pad_e = _EP - _E
    pad_i = jnp.arange(pad_e, dtype=jnp.int32)
    src_p = jnp.concatenate([src, pad_i % _N]).reshape(_EP // _C, _C)
    dst_p = jnp.concatenate([dst, _N + pad_i % (_NP - _N)]).reshape(
        _EP // _C, _C)
    h = jnp.zeros((_NP, _D), jnp.float32).at[:_N].set(x)

    wp_pad = jnp.zeros((_D, _D), jnp.float32).at[:, :_OUT].set(Wp)
    bp_pad = jnp.zeros((1, _D), jnp.float32).at[0, :_OUT].set(bp)

    (pd_flat,) = _deg(dst_p)
    pd = pd_flat.reshape(2, _NP)
    out = None
    for i in range(_L):
        (ph,) = _agg(h, src_p, dst_p)
        p = ph.reshape(2, _NP, _D)
        ws = Ws[i]
        wn = Wn[i]
        b = bs[i].reshape(1, _D)
        if i < _L - 1:
            h = _mid_layer_tc(h, p, pd, ws, wn, b)
        else:
            out = _last_layer_tc(h, p, pd, ws, wn, b, wp_pad, bp_pad)
    return out[:_N, :_OUT]


# speedup vs baseline: 1.1806x; 1.0051x over previous
import functools

import jax
import jax.numpy as jnp
from jax import lax
from jax.experimental import pallas as pl
from jax.experimental.pallas import tpu as pltpu
from jax.experimental.pallas import tpu_sc as plsc

_N = 10000
_D = 128
_OUT = 26
_L = 4
_E = 320000

_NP = 10240
_RPT = _NP // 16
_EP = 327680
_EPT = _EP // 32
_C = 128
_NCHUNK = _EPT // _C
_NB = 2

_mesh = plsc.VectorSubcoreMesh(core_axis_name="c", subcore_axis_name="s")


def _fill(ref, rows, cols, val):
    v = jnp.full((16,), val, jnp.float32)

    def row(i, carry):
        for j in range(cols // 16):
            ref[i, pl.ds(j * 16, 16)] = v
        return carry

    lax.fori_loop(0, rows, row, 0)


def _make_agg():
    out_type = [jax.ShapeDtypeStruct((2 * _NP, _D), jnp.float32)]
    scratch = [
        pltpu.VMEM((_NB, _C), jnp.int32),
        pltpu.VMEM((_C,), jnp.int32),
        pltpu.VMEM((_NCHUNK, _C), jnp.int32),
        pltpu.VMEM((_NB, _C, _D), jnp.float32),
        pltpu.VMEM_SHARED((_NP, _D), jnp.float32),
    ] + [pltpu.SemaphoreType.DMA] * _NB

    @functools.partial(pl.kernel, out_type=out_type, mesh=_mesh,
                       scratch_types=scratch,
                       compiler_params=pltpu.CompilerParams(
                           needs_layout_passes=False))
    def agg(h_hbm, edges_hbm, out_h, src_c, dst_c, pk, rows, acc_h, *gsems):
        cid = lax.axis_index("c")
        sid = lax.axis_index("s")
        wid = sid * 2 + cid

        c0 = pl.multiple_of(wid * _NCHUNK, _NCHUNK)
        pltpu.sync_copy(edges_hbm.at[pl.ds(c0, _NCHUNK)], pk)

        def unpack_src(j, b):
            for k in range(_C // 16):
                w = pk[j, pl.ds(k * 16, 16)]
                src_c[b, pl.ds(k * 16, 16)] = jnp.bitwise_and(w, 0xFFFF)

        def unpack_dst(j):
            for k in range(_C // 16):
                w = pk[j, pl.ds(k * 16, 16)]
                dst_c[pl.ds(k * 16, 16)] = lax.shift_right_logical(w, 16)

        _fill(rows.at[0], _C, _D, 0.0)
        r0 = pl.multiple_of(sid * _RPT, _RPT)
        for k in range(_RPT // _C):
            pltpu.sync_copy(rows.at[0], acc_h.at[pl.ds(r0 + k * _C, _C)])

        for b in range(2):
            unpack_src(b, b)
            pltpu.async_copy(h_hbm.at[src_c.at[b]], rows.at[b], gsems[b])
        plsc.subcore_barrier()

        def do_chunk(j, b, fire_next):
            unpack_dst(j)
            pltpu.make_async_copy(h_hbm.at[src_c.at[b]], rows.at[b],
                                  gsems[b]).wait()
            if fire_next:
                unpack_src(j + 2, b)
            pltpu.sync_copy(rows.at[b], acc_h.at[dst_c], add=True)
            if fire_next:
                pltpu.async_copy(h_hbm.at[src_c.at[b]], rows.at[b], gsems[b])

        def pair(g, carry):
            for b in range(2):
                do_chunk(g * 2 + b, b, True)
            return carry

        lax.fori_loop(0, _NCHUNK // 2 - 1, pair, 0)
        for b in range(2):
            do_chunk(_NCHUNK - 2 + b, b, False)
        plsc.subcore_barrier()

        o0 = pl.multiple_of(cid * _NP + sid * _RPT, _RPT)
        pltpu.sync_copy(acc_h.at[pl.ds(r0, _RPT)], out_h.at[pl.ds(o0, _RPT)])

    return agg


def _make_deg():
    out_type = [jax.ShapeDtypeStruct((2 * _NP,), jnp.float32)]
    scratch = [
        pltpu.VMEM((_NCHUNK, _C), jnp.int32),
        pltpu.VMEM((_NP,), jnp.float32),
        pltpu.VMEM((16, _RPT), jnp.float32),
        pltpu.VMEM_SHARED((16, _NP), jnp.float32),
    ]

    @functools.partial(pl.kernel, out_type=out_type, mesh=_mesh,
                       scratch_types=scratch,
                       compiler_params=pltpu.CompilerParams(
                           needs_layout_passes=False))
    def deg(dst_hbm, out_d, dst_l, deg_l, stage, deg_sh):
        cid = lax.axis_index("c")
        sid = lax.axis_index("s")
        wid = sid * 2 + cid

        c0 = pl.multiple_of(wid * _NCHUNK, _NCHUNK)
        pltpu.sync_copy(dst_hbm.at[pl.ds(c0, _NCHUNK)], dst_l)
        z = jnp.zeros((16,), jnp.float32)

        def zrow(i, carry):
            deg_l[pl.ds(i * 16, 16)] = z
            return carry

        lax.fori_loop(0, _NP // 16, zrow, 0)
        ones = jnp.ones((16,), jnp.float32)

        def chunk(j, carry):
            for k in range(_C // 16):
                iv = dst_l[j, pl.ds(k * 16, 16)]
                plsc.addupdate_scatter(deg_l, [iv], ones)
            return carry

        lax.fori_loop(0, _NCHUNK, chunk, 0)
        pltpu.sync_copy(deg_l, deg_sh.at[sid])
        plsc.subcore_barrier()
        r0 = pl.multiple_of(sid * _RPT, _RPT)
        for t in range(16):
            pltpu.sync_copy(deg_sh.at[t, pl.ds(r0, _RPT)], stage.at[t])

        def srow(i, carry):
            s = jnp.zeros((16,), jnp.float32)
            for t in range(16):
                s = s + stage[t, pl.ds(i * 16, 16)]
            deg_l[pl.ds(i * 16, 16)] = s
            return carry

        lax.fori_loop(0, _RPT // 16, srow, 0)
        o0 = pl.multiple_of(cid * _NP + sid * _RPT, _RPT)
        pltpu.sync_copy(deg_l.at[pl.ds(0, _RPT)], out_d.at[pl.ds(o0, _RPT)])

    return deg


_agg = _make_agg()
_deg = _make_deg()

_BR = 640


def _mid_body(h_ref, p_ref, d_ref, ws_ref, wn_ref, b_ref, o_ref):
    deg = jnp.maximum(d_ref[0] + d_ref[1], 1.0)
    neigh = (p_ref[0] + p_ref[1]) * (1.0 / deg)[:, None]
    acc = jnp.dot(h_ref[...], ws_ref[...], preferred_element_type=jnp.float32)
    acc = acc + jnp.dot(neigh, wn_ref[...],
                        preferred_element_type=jnp.float32)
    o_ref[...] = jnp.maximum(acc + b_ref[...], 0.0)


def _last_body(h_ref, p_ref, d_ref, ws_ref, wn_ref, b_ref, wp_ref, bp_ref,
               o_ref):
    deg = jnp.maximum(d_ref[0] + d_ref[1], 1.0)
    neigh = (p_ref[0] + p_ref[1]) * (1.0 / deg)[:, None]
    acc = jnp.dot(h_ref[...], ws_ref[...], preferred_element_type=jnp.float32)
    acc = acc + jnp.dot(neigh, wn_ref[...],
                        preferred_element_type=jnp.float32)
    h4 = jnp.maximum(acc + b_ref[...], 0.0)
    o_ref[...] = jnp.dot(h4, wp_ref[...],
                         preferred_element_type=jnp.float32) + bp_ref[...]


_WSPEC = [
    pl.BlockSpec((_D, _D), lambda i: (0, 0)),
    pl.BlockSpec((_D, _D), lambda i: (0, 0)),
    pl.BlockSpec((1, _D), lambda i: (0, 0)),
]
_HSPEC = pl.BlockSpec((_BR, _D), lambda i: (i, 0))
_PSPEC = pl.BlockSpec((2, _BR, _D), lambda i: (0, i, 0))
_DSPEC = pl.BlockSpec((2, _BR), lambda i: (0, i))


def _mid_layer_tc(h, p, d, ws, wn, b):
    return pl.pallas_call(
        _mid_body,
        grid=(_NP // _BR,),
        in_specs=[_HSPEC, _PSPEC, _DSPEC] + _WSPEC,
        out_specs=_HSPEC,
        out_shape=jax.ShapeDtypeStruct((_NP, _D), jnp.float32),
    )(h, p, d, ws, wn, b)


def _last_layer_tc(h, p, d, ws, wn, b, wp, bp):
    return pl.pallas_call(
        _last_body,
        grid=(_NP // _BR,),
        in_specs=[_HSPEC, _PSPEC, _DSPEC] + _WSPEC + [
            pl.BlockSpec((_D, _D), lambda i: (0, 0)),
            pl.BlockSpec((1, _D), lambda i: (0, 0)),
        ],
        out_specs=_HSPEC,
        out_shape=jax.ShapeDtypeStruct((_NP, _D), jnp.float32),
    )(h, p, d, ws, wn, b, wp, bp)


def kernel(x, edge_index, Ws, Wn, bs, Wp, bp):
    src = edge_index[0]
    dst = edge_index[1]
    pad_e = _EP - _E
    pad_i = jnp.arange(pad_e, dtype=jnp.int32)
    src_p = jnp.concatenate([src, pad_i % _N])
    dst_p = jnp.concatenate([dst, _N + pad_i % (_NP - _N)])
    edges_p = jnp.bitwise_or(src_p, dst_p << 16).reshape(_EP // _C, _C)
    dst_p = dst_p.reshape(_EP // _C, _C)
    h = jnp.zeros((_NP, _D), jnp.float32).at[:_N].set(x)

    wp_pad = jnp.zeros((_D, _D), jnp.float32).at[:, :_OUT].set(Wp)
    bp_pad = jnp.zeros((1, _D), jnp.float32).at[0, :_OUT].set(bp)

    (pd_flat,) = _deg(dst_p)
    pd = pd_flat.reshape(2, _NP)
    out = None
    for i in range(_L):
        (ph,) = _agg(h, edges_p)
        p = ph.reshape(2, _NP, _D)
        ws = Ws[i]
        wn = Wn[i]
        b = bs[i].reshape(1, _D)
        if i < _L - 1:
            h = _mid_layer_tc(h, p, pd, ws, wn, b)
        else:
            out = _last_layer_tc(h, p, pd, ws, wn, b, wp_pad, bp_pad)
    return out[:_N, :_OUT]
